# baseline (speedup 1.0000x reference)
import functools

import jax
import jax.numpy as jnp
from jax import lax
from jax.experimental import pallas as pl
from jax.experimental.pallas import tpu as pltpu
from jax.experimental.pallas import tpu_sc as plsc

Nc = 4096
Nt = 4096
NNZ = 167772
D = 1024

NG = 8
DG = D // NG
N_TILES = 16
G = 128
K = 2
CHUNKS = 84
NNZ_PAD = N_TILES * CHUNKS * G
ACC_ROWS = Nc + 8
RPT = Nc // N_TILES
N_PASS = 4
LANES = 16

assert CHUNKS % K == 0


def _sc_body(mat_ref, ridx_ref, cidx_ref, zeros_ref, out_ref,
             ridx_v, cidx_v, vals, acc, gsem, ssem):
    c = lax.axis_index("c")
    s = lax.axis_index("s")

    pltpu.sync_copy(ridx_ref.at[s], ridx_v)

    for p in range(N_PASS):
        g = c * N_PASS + p
        pltpu.sync_copy(cidx_ref.at[s], cidx_v)

        def fix(j, carry):
            for t in range(G // LANES):
                sl = pl.ds(t * LANES, LANES)
                cidx_v[j, sl] = cidx_v[j, sl] * NG + g
            return carry

        lax.fori_loop(0, CHUNKS, fix, 0)

        pltpu.sync_copy(zeros_ref, acc.at[pl.ds(s * RPT, RPT)])
        plsc.subcore_barrier()

        def step(k, carry):
            descs = []
            for b in range(K):
                j = K * k + b
                descs.append(pltpu.async_copy(
                    mat_ref.at[cidx_v.at[j]], vals.at[b], gsem))
            for d in descs:
                d.wait()

            def sstep(b, carry2):
                pltpu.async_copy(vals.at[b], acc.at[ridx_v.at[K * k + b]],
                                 ssem, add=True).wait()
                return carry2

            lax.fori_loop(0, K, sstep, 0)
            return carry

        lax.fori_loop(0, CHUNKS // K, step, 0)
        plsc.subcore_barrier()
        pltpu.sync_copy(acc.at[pl.ds(s * RPT, RPT)],
                        out_ref.at[pl.ds(s * RPT, RPT), g])


_sc_call = functools.partial(
    pl.kernel,
    out_type=jax.ShapeDtypeStruct((Nc, NG, DG), jnp.float32),
    mesh=plsc.VectorSubcoreMesh(core_axis_name="c", subcore_axis_name="s"),
    scratch_types=[
        pltpu.VMEM((CHUNKS, G), jnp.int32),
        pltpu.VMEM((CHUNKS, G), jnp.int32),
        pltpu.VMEM((K, G, DG), jnp.float32),
        pltpu.VMEM_SHARED((ACC_ROWS, DG), jnp.float32),
        pltpu.SemaphoreType.DMA,
        pltpu.SemaphoreType.DMA,
    ],
)(_sc_body)


def kernel(mat, row, col):
    pad = NNZ_PAD - NNZ
    row_p = jnp.concatenate([row, jnp.full((pad,), Nc, jnp.int32)])
    col_p = jnp.concatenate([col, jnp.zeros((pad,), jnp.int32)])
    ridx = row_p.reshape(N_TILES, CHUNKS, G)
    cidx = col_p.reshape(N_TILES, CHUNKS, G)
    mat_r = mat.reshape(Nt * NG, DG)
    zeros = jnp.zeros((RPT, DG), jnp.float32)
    out3 = _sc_call(mat_r, ridx, cidx, zeros)
    return out3.reshape(Nc, D)


# speedup vs baseline: 1.9482x; 1.9482x over previous
import functools

import jax
import jax.numpy as jnp
from jax import lax
from jax.experimental import pallas as pl
from jax.experimental.pallas import tpu as pltpu
from jax.experimental.pallas import tpu_sc as plsc

Nc = 4096
Nt = 4096
NNZ = 167772
D = 1024

NG = 8
DG = D // NG
N_TILES = 16
G = 128
CHUNKS = -(-NNZ // (N_TILES * G))
NNZ_PAD = N_TILES * CHUNKS * G
ACC_ROWS = Nc + 8
RPT = Nc // N_TILES
N_PASS = 4


def _sc_body(mat_ref, ridx_ref, colg_ref, zeros_ref, out_ref,
             ridx_v, cidx_v, vals_v, acc, gsem, ssem):
    c = lax.axis_index("c")
    s = lax.axis_index("s")

    pltpu.sync_copy(ridx_ref.at[s], ridx_v)

    for p in range(N_PASS):
        g = c * N_PASS + p
        pltpu.sync_copy(colg_ref.at[g, s], cidx_v)
        pltpu.sync_copy(zeros_ref, acc.at[pl.ds(s * RPT, RPT)])
        plsc.subcore_barrier()

        def step(j, carry):
            pltpu.async_copy(mat_ref.at[cidx_v.at[j]], vals_v, gsem).wait()
            pltpu.async_copy(vals_v, acc.at[ridx_v.at[j]], ssem,
                             add=True).wait()
            return carry

        lax.fori_loop(0, CHUNKS, step, 0)
        plsc.subcore_barrier()
        pltpu.sync_copy(acc.at[pl.ds(s * RPT, RPT)],
                        out_ref.at[pl.ds(s * RPT, RPT), g])


_sc_call = functools.partial(
    pl.kernel,
    out_type=jax.ShapeDtypeStruct((Nc, NG, DG), jnp.float32),
    mesh=plsc.VectorSubcoreMesh(core_axis_name="c", subcore_axis_name="s"),
    scratch_types=[
        pltpu.VMEM((CHUNKS, G), jnp.int32),
        pltpu.VMEM((CHUNKS, G), jnp.int32),
        pltpu.VMEM((G, DG), jnp.float32),
        pltpu.VMEM_SHARED((ACC_ROWS, DG), jnp.float32),
        pltpu.SemaphoreType.DMA,
        pltpu.SemaphoreType.DMA,
    ],
)(_sc_body)


def kernel(mat, row, col):
    pad = NNZ_PAD - NNZ
    row_p = jnp.concatenate([row, jnp.full((pad,), Nc, jnp.int32)])
    col_p = jnp.concatenate([col, jnp.zeros((pad,), jnp.int32)])
    ridx = row_p.reshape(N_TILES, CHUNKS, G)
    gs = jnp.arange(NG, dtype=jnp.int32)[:, None]
    colg = (col_p[None, :] * NG + gs).reshape(NG, N_TILES, CHUNKS, G)
    mat_r = mat.reshape(Nt * NG, DG)
    zeros = jnp.zeros((RPT, DG), jnp.float32)
    out3 = _sc_call(mat_r, ridx, colg, zeros)
    return out3.reshape(Nc, D)
